# baseline (speedup 1.0000x reference)
import functools

import jax
import jax.numpy as jnp
from jax import lax
from jax.experimental import pallas as pl
from jax.experimental.pallas import tpu as pltpu
from jax.experimental.pallas import tpu_sc as plsc

ROWS, D = 8192, 768
NC, NS = 2, 16
NW = NC * NS
ROWS_PER = ROWS // NW
CHUNK = 32
NCHUNK = ROWS_PER // CHUNK
NBUF = 4

_mesh = plsc.VectorSubcoreMesh(core_axis_name="c", subcore_axis_name="s")


@functools.partial(
    pl.kernel,
    mesh=_mesh,
    out_type=jax.ShapeDtypeStruct((ROWS, D), jnp.float32),
    scratch_types=(
        [pltpu.VMEM((CHUNK, D), jnp.float32) for _ in range(NBUF)]
        + [pltpu.SemaphoreType.DMA for _ in range(2 * NBUF)]
    ),
)
def _sc_copy(w_hbm, out_hbm, *scratch):
    bufs = scratch[:NBUF]
    gsem = scratch[NBUF : 2 * NBUF]
    ssem = scratch[2 * NBUF :]
    wid = lax.axis_index("s") * NC + lax.axis_index("c")
    base = wid * ROWS_PER

    gathers = [None] * NCHUNK
    scatters = [None] * NCHUNK

    def g(i):
        b = i % NBUF
        return pltpu.async_copy(
            w_hbm.at[pl.ds(base + i * CHUNK, CHUNK)], bufs[b], gsem[b]
        )

    def s(i):
        b = i % NBUF
        return pltpu.async_copy(
            bufs[b], out_hbm.at[pl.ds(base + i * CHUNK, CHUNK)], ssem[b]
        )

    for i in range(NBUF):
        gathers[i] = g(i)
    for i in range(NCHUNK):
        if 0 < i and i - 1 + NBUF < NCHUNK:
            scatters[i - 1].wait()
            gathers[i - 1 + NBUF] = g(i - 1 + NBUF)
        gathers[i].wait()
        scatters[i] = s(i)
    for i in range(max(0, NCHUNK - NBUF), NCHUNK):
        scatters[i].wait()


def kernel(x, W):
    del x
    return _sc_copy(W)


# speedup vs baseline: 2.3397x; 2.3397x over previous
import jax
import jax.numpy as jnp
from jax.experimental import pallas as pl

_MAX_BLOCK_ROWS = 4096


def _copy_body(w_ref, o_ref):
    o_ref[...] = w_ref[...]


def kernel(x, W):
    rows = x.shape[0]
    d = W.shape[1]
    block = min(rows, _MAX_BLOCK_ROWS)
    while rows % block:
        block -= 1
    return pl.pallas_call(
        _copy_body,
        grid=(rows // block,),
        in_specs=[pl.BlockSpec((block, d), lambda i: (i, 0))],
        out_specs=pl.BlockSpec((block, d), lambda i: (i, 0)),
        out_shape=jax.ShapeDtypeStruct((rows, d), W.dtype),
    )(W)
